# Initial kernel scaffold; baseline (speedup 1.0000x reference)
import jax
import jax.numpy as jnp
from jax.experimental import pallas as pl


def kernel(xyz, xyz_batch_cnt, new_xyz, new_xyz_batch_cnt, features, W1, bn1_g, bn1_b, W2, bn2_g, bn2_b):
    raise NotImplementedError("write your pallas kernel here")


# speedup vs baseline: 3.1562x; 3.1562x over previous
import functools

import jax
import jax.numpy as jnp
from jax import lax
from jax.experimental import pallas as pl
from jax.experimental.pallas import tpu as pltpu
from jax.experimental.pallas import tpu_sc as plsc

RADIUS2 = 1.0
NSAMPLE = 32
EPS = 1e-5
BM = 128
BK = 256


def _ballquery_body(nref, xref, idxref, rxref, ryref, rzref, cntref):
    nx = nref[:, 0:1]
    ny = nref[:, 1:2]
    nz = nref[:, 2:3]
    xx = xref[0:1, :]
    xy = xref[1:2, :]
    xz = xref[2:3, :]
    d2 = ((nx - xx) * (nx - xx) + (ny - xy) * (ny - xy)
          + (nz - xz) * (nz - xz))
    within = d2 <= RADIUS2
    cntref[...] = jnp.sum(within.astype(jnp.float32), axis=1, keepdims=True) * (1.0 / NSAMPLE)

    iota = lax.broadcasted_iota(jnp.int32, d2.shape, 1)
    iota_s = lax.broadcasted_iota(jnp.int32, (d2.shape[0], NSAMPLE), 1)
    dm0 = jnp.where(within, d2, jnp.inf)
    zero_s = jnp.zeros((d2.shape[0], NSAMPLE), jnp.float32)
    acc0 = (jnp.zeros((d2.shape[0], NSAMPLE), jnp.int32), zero_s, zero_s, zero_s)
    big = jnp.int32(2 ** 30)

    def body(k, carry):
        dm, (ai, ax, ay, az) = carry
        v = jnp.min(dm, axis=1, keepdims=True)
        a = jnp.min(jnp.where(dm <= v, iota, big), axis=1, keepdims=True)
        eqa = (iota == a).astype(jnp.float32)
        selx = jnp.sum(xx * eqa, axis=1, keepdims=True)
        sely = jnp.sum(xy * eqa, axis=1, keepdims=True)
        selz = jnp.sum(xz * eqa, axis=1, keepdims=True)
        live = (v < jnp.inf) | (k == 0)
        slot = iota_s == k
        ai = jnp.where(slot, jnp.where(live, a, ai[:, 0:1]), ai)
        ax = jnp.where(slot, jnp.where(live, selx, ax[:, 0:1]), ax)
        ay = jnp.where(slot, jnp.where(live, sely, ay[:, 0:1]), ay)
        az = jnp.where(slot, jnp.where(live, selz, az[:, 0:1]), az)
        dm = jnp.where(iota == a, jnp.inf, dm)
        return dm, (ai, ax, ay, az)

    _, (ai, ax, ay, az) = lax.fori_loop(0, NSAMPLE, body, (dm0, acc0))
    idxref[...] = ai
    rxref[...] = ax - nx
    ryref[...] = ay - ny
    rzref[...] = az - nz


def _ball_query(new_xyz, xyzT, m, npad):
    grid = m // BM
    blk = lambda i: (i, 0)
    return pl.pallas_call(
        _ballquery_body,
        grid=(grid,),
        in_specs=[
            pl.BlockSpec((BM, 3), blk),
            pl.BlockSpec((3, npad), lambda i: (0, 0)),
        ],
        out_specs=[
            pl.BlockSpec((BM, NSAMPLE), blk),
            pl.BlockSpec((BM, NSAMPLE), blk),
            pl.BlockSpec((BM, NSAMPLE), blk),
            pl.BlockSpec((BM, NSAMPLE), blk),
            pl.BlockSpec((BM, 1), blk),
        ],
        out_shape=[
            jax.ShapeDtypeStruct((m, NSAMPLE), jnp.int32),
            jax.ShapeDtypeStruct((m, NSAMPLE), jnp.float32),
            jax.ShapeDtypeStruct((m, NSAMPLE), jnp.float32),
            jax.ShapeDtypeStruct((m, NSAMPLE), jnp.float32),
            jax.ShapeDtypeStruct((m, 1), jnp.float32),
        ],
    )(new_xyz, xyzT)


def _gather_sc(tabf, idx_flat, e, c):
    info = plsc.get_sparse_core_info()
    nw = info.num_cores * info.num_subcores
    b_per_w = e // nw
    chunk = 512
    nchunk = b_per_w // chunk
    mesh = plsc.VectorSubcoreMesh(core_axis_name="c", subcore_axis_name="s")

    @functools.partial(
        pl.kernel,
        mesh=mesh,
        out_type=jax.ShapeDtypeStruct((e, c), jnp.float32),
        scratch_types=[
            pltpu.VMEM((chunk,), jnp.int32),
            pltpu.VMEM((chunk, c), jnp.float32),
            pltpu.SemaphoreType.DMA,
        ],
    )
    def k(tabf_hbm, idx_hbm, outf_hbm, idx_v, rows_v, sem):
        wid = lax.axis_index("s") * info.num_cores + lax.axis_index("c")
        base = wid * b_per_w

        def body(ci, _):
            off = base + ci * chunk
            pltpu.sync_copy(idx_hbm.at[pl.ds(off, chunk)], idx_v)
            for j in range(chunk // 128):
                sl = pl.ds(j * 128, 128)
                pltpu.async_copy(tabf_hbm.at[idx_v.at[sl]], rows_v.at[sl], sem).wait()
            pltpu.sync_copy(rows_v, outf_hbm.at[pl.ds(off, chunk)])
            return 0

        lax.fori_loop(0, nchunk, body, 0)

    return k(tabf, idx_flat)


def _geo_h(gref, w1ref):
    rx = gref[:, 0:1]
    ry = gref[:, 1:2]
    rz = gref[:, 2:3]
    d2g = rx * rx + ry * ry + rz * rz
    dist = jnp.sqrt(jnp.maximum(d2g, 1e-12))
    cnt = gref[:, 3:4]
    h = (rx * w1ref[0:1, :] + ry * w1ref[1:2, :] + rz * w1ref[2:3, :]
         + dist * w1ref[3:4, :] + cnt * w1ref[4:5, :])
    return h


def _stats_body(gref, w1ref, sref):
    h = _geo_h(gref, w1ref)

    @pl.when(pl.program_id(0) == 0)
    def _():
        sref[...] = jnp.zeros_like(sref)

    sref[0:1, :] = sref[0:1, :] + jnp.sum(h, axis=0, keepdims=True)
    sref[1:2, :] = sref[1:2, :] + jnp.sum(h * h, axis=0, keepdims=True)


def _bn1_stats(rels, w1, e, hid):
    be = BK * NSAMPLE
    grid = e // be
    return pl.pallas_call(
        _stats_body,
        grid=(grid,),
        in_specs=[
            pl.BlockSpec((be, 4), lambda i: (i, 0)),
            pl.BlockSpec((5, hid), lambda i: (0, 0)),
        ],
        out_specs=pl.BlockSpec((2, hid), lambda i: (0, 0)),
        out_shape=jax.ShapeDtypeStruct((2, hid), jnp.float32),
    )(rels, w1)


def _mlp_body(e_edges, gref, s1ref, w1ref, g1ref, b1ref, w2ref, g2ref):
    h = _geo_h(gref, w1ref)
    inv_e = 1.0 / e_edges
    mu = s1ref[0:1, :] * inv_e
    var = s1ref[1:2, :] * inv_e - mu * mu
    hn = (h - mu) * lax.rsqrt(var + EPS) * g1ref[...] + b1ref[...]
    hn = jnp.maximum(hn, 0.0)
    g2ref[...] = jnp.dot(hn, w2ref[...], preferred_element_type=jnp.float32)


def _mlp(rels, s1, w1, g1, b1, w2, e, hid, c):
    be = BK * NSAMPLE
    grid = e // be
    return pl.pallas_call(
        functools.partial(_mlp_body, float(e)),
        grid=(grid,),
        in_specs=[
            pl.BlockSpec((be, 4), lambda i: (i, 0)),
            pl.BlockSpec((2, hid), lambda i: (0, 0)),
            pl.BlockSpec((5, hid), lambda i: (0, 0)),
            pl.BlockSpec((1, hid), lambda i: (0, 0)),
            pl.BlockSpec((1, hid), lambda i: (0, 0)),
            pl.BlockSpec((hid, c), lambda i: (0, 0)),
        ],
        out_specs=pl.BlockSpec((be, c), lambda i: (i, 0)),
        out_shape=jax.ShapeDtypeStruct((e, c), jnp.float32),
    )(rels, s1, w1, g1, b1, w2)


def _att_body(fgref, g3ref, aggref, s2ref):
    c = fgref.shape[1]
    kp_per_q = c // NSAMPLE
    nq = g3ref.shape[1]

    @pl.when(pl.program_id(0) == 0)
    def _():
        s2ref[...] = jnp.zeros_like(s2ref)

    def body(j, carry):
        ssum, ssq = carry
        w = g3ref[:, pl.ds(j, 1), :].reshape(c, c)
        fgs = fgref[pl.ds(j * c, c), :]
        p = fgs * w.T
        a = p.reshape(kp_per_q, NSAMPLE, c).sum(axis=1)
        aggref[pl.ds(j * kp_per_q, kp_per_q), :] = a
        return (ssum + jnp.sum(a, axis=0, keepdims=True),
                ssq + jnp.sum(a * a, axis=0, keepdims=True))

    zero = jnp.zeros((1, c), jnp.float32)
    ssum, ssq = lax.fori_loop(0, nq, body, (zero, zero))
    s2ref[0:1, :] = s2ref[0:1, :] + ssum
    s2ref[1:2, :] = s2ref[1:2, :] + ssq


def _att(fg, g3, m, e, c):
    be = BK * NSAMPLE
    grid = e // be
    nq = be // c
    return pl.pallas_call(
        _att_body,
        grid=(grid,),
        in_specs=[
            pl.BlockSpec((be, c), lambda i: (i, 0)),
            pl.BlockSpec((c, nq, c), lambda i: (0, i, 0)),
        ],
        out_specs=[
            pl.BlockSpec((BK, c), lambda i: (i, 0)),
            pl.BlockSpec((2, c), lambda i: (0, 0)),
        ],
        out_shape=[
            jax.ShapeDtypeStruct((m, c), jnp.float32),
            jax.ShapeDtypeStruct((2, c), jnp.float32),
        ],
    )(fg, g3)


def _bn2_body(m_rows, aggref, s2ref, gref, bref, outref):
    inv_m = 1.0 / m_rows
    mu = s2ref[0:1, :] * inv_m
    var = s2ref[1:2, :] * inv_m - mu * mu
    x = (aggref[...] - mu) * lax.rsqrt(var + EPS) * gref[...] + bref[...]
    outref[...] = jnp.maximum(x, 0.0)


def _bn2(agg, s2, g2, b2, m, c):
    return pl.pallas_call(
        functools.partial(_bn2_body, float(m)),
        grid=(1,),
        in_specs=[
            pl.BlockSpec((m, c), lambda i: (0, 0)),
            pl.BlockSpec((2, c), lambda i: (0, 0)),
            pl.BlockSpec((1, c), lambda i: (0, 0)),
            pl.BlockSpec((1, c), lambda i: (0, 0)),
        ],
        out_specs=pl.BlockSpec((m, c), lambda i: (0, 0)),
        out_shape=jax.ShapeDtypeStruct((m, c), jnp.float32),
    )(agg, s2, g2, b2)


def kernel(xyz, xyz_batch_cnt, new_xyz, new_xyz_batch_cnt, features,
           W1, bn1_g, bn1_b, W2, bn2_g, bn2_b):
    n, c = features.shape
    m = new_xyz.shape[0]
    hid = W1.shape[1]
    e = m * NSAMPLE
    npad = ((n + 1023) // 1024) * 1024

    xyzT = jnp.concatenate(
        [xyz.T, jnp.full((3, npad - n), 1e9, jnp.float32)], axis=1)
    idx, rx, ry, rz, cnt = _ball_query(new_xyz, xyzT, m, npad)

    fg = _gather_sc(features, idx.reshape(e), e, c)

    rels = jnp.concatenate(
        [rx.reshape(e, 1), ry.reshape(e, 1), rz.reshape(e, 1),
         jnp.repeat(cnt, NSAMPLE, axis=0).reshape(e, 1)], axis=1)

    s1 = _bn1_stats(rels, W1, e, hid)
    g2 = _mlp(rels, s1, W1, bn1_g.reshape(1, hid),
              bn1_b.reshape(1, hid), W2, e, hid, c)
    g3 = g2.reshape(c, e // c, c)
    agg, s2 = _att(fg, g3, m, e, c)
    out = _bn2(agg, s2, bn2_g.reshape(1, c), bn2_b.reshape(1, c), m, c)
    return new_xyz, out
